# baseline (speedup 1.0000x reference)
import functools

import jax
import jax.numpy as jnp
from jax import lax
from jax.experimental import pallas as pl
from jax.experimental.pallas import tpu as pltpu
from jax.experimental.pallas import tpu_sc as plsc

B = 16384
K = 32

_NC = 2
_NS = 16
_NW = _NC * _NS
_BPW = B // _NW
_CH = 128
_NCHUNK = _BPW // _CH

@functools.cache
def _make_gather_sc():
    mesh = plsc.VectorSubcoreMesh(core_axis_name="c", subcore_axis_name="s")

    @functools.partial(
        pl.kernel,
        mesh=mesh,
        compiler_params=pltpu.CompilerParams(use_tc_tiling_on_sc=False),
        out_type=[
            jax.ShapeDtypeStruct((B, K), jnp.float32),
            jax.ShapeDtypeStruct((B, K), jnp.float32),
        ],
        scratch_types=[
            pltpu.VMEM((_NCHUNK, _CH), jnp.int32),
            pltpu.VMEM((_NCHUNK, _CH), jnp.int32),
            pltpu.VMEM((_BPW, K), jnp.float32),
            pltpu.VMEM((_BPW, K), jnp.float32),
            pltpu.SemaphoreType.DMA,
            pltpu.SemaphoreType.DMA,
        ],
    )
    def _gather_sc(w_hbm, h_hbm, ui_hbm, vi_hbm, u_out, v_out,
                   ui_v, vi_v, u_rows, v_rows, su, sv):
        wid = lax.axis_index("s") * _NC + lax.axis_index("c")
        base = wid * _BPW
        pltpu.sync_copy(ui_hbm.at[wid], ui_v)
        pltpu.sync_copy(vi_hbm.at[wid], vi_v)
        copies = []
        for j in range(_NCHUNK):
            copies.append(pltpu.async_copy(
                w_hbm.at[ui_v.at[j]], u_rows.at[pl.ds(j * _CH, _CH)], su))
            copies.append(pltpu.async_copy(
                h_hbm.at[vi_v.at[j]], v_rows.at[pl.ds(j * _CH, _CH)], sv))
        for c in copies:
            c.wait()
        pltpu.sync_copy(u_rows, u_out.at[pl.ds(base, _BPW)])
        pltpu.sync_copy(v_rows, v_out.at[pl.ds(base, _BPW)])

    return _gather_sc


def _mlp_body(u_ref, v_ref, a_ref, c_ref, w2_ref, b2_ref, o_ref):
    h = jnp.dot(u_ref[...], a_ref[...], preferred_element_type=jnp.float32)
    h = h + jnp.dot(v_ref[...], c_ref[...], preferred_element_type=jnp.float32)
    h = jnp.maximum(h, 0.0)
    logit = jnp.sum(h * w2_ref[...], axis=1) + b2_ref[...]
    o_ref[...] = jax.nn.sigmoid(logit)


def _mlp_tc(u, v, a, c, w2, b2):
    return pl.pallas_call(
        _mlp_body,
        out_shape=jax.ShapeDtypeStruct((B,), jnp.float32),
    )(u, v, a, c, w2, b2)


def kernel(x, W, H, W1, W2, b2):
    ui = x[:, 0].astype(jnp.int32).reshape(_NW, _NCHUNK, _CH)
    vi = x[:, 1].astype(jnp.int32).reshape(_NW, _NCHUNK, _CH)
    u, v = _make_gather_sc()(W, H, ui, vi)
    a = jnp.transpose(W1[:, :K])
    c = jnp.transpose(W1[:, K:])
    return _mlp_tc(u, v, a, c, W2, b2)
